# Initial kernel scaffold; baseline (speedup 1.0000x reference)
import jax
import jax.numpy as jnp
from jax.experimental import pallas as pl


def kernel(x, h_in, node_adj, edge_adj, pn_w1, pn_b1, pn_w2, pn_b2, pn_w3, pn_b3, it1_w, it1_b, it2_w, it2_b, gru_wr, gru_ur, gru_br, gru_wz, gru_uz, gru_bz, gru_wn, gru_un, gru_bn, out_node_w, out_node_b, out_edge_w, out_edge_b):
    raise NotImplementedError("write your pallas kernel here")


# speedup vs baseline: 1.0269x; 1.0269x over previous
import jax
import jax.numpy as jnp
from jax.experimental import pallas as pl
from jax.experimental.pallas import tpu as pltpu

N_NEW = 2048
N_OLD = 4096
T = N_OLD + N_NEW
NHID = 64

_ROWS = 256
_NTILE = T // _ROWS


def _dot(a, b):
    return jnp.dot(a.astype(jnp.bfloat16), b.astype(jnp.bfloat16),
                   preferred_element_type=jnp.float32)


def _diag_new_body(node_ref, out_ref):
    blk = node_ref[...]
    ii = jax.lax.broadcasted_iota(jnp.int32, (128, 128), 0)
    jj = jax.lax.broadcasted_iota(jnp.int32, (128, 128), 1)
    d = jnp.sum(jnp.where(ii == jj, blk, 0.0), axis=0, keepdims=True)
    out_ref[...] = d.reshape(1, 1, 128)


def _preproc_body(x_ref, w1_ref, b1_ref, w2_ref, b2_ref, w3_ref, b3_ref,
                  it1a_ref, it1b_ref, it1_bias_ref, it2_ref, it2_bias_ref,
                  dnew_ref, out_ref):
    eps = 1e-5
    w1 = w1_ref[...]
    parts = []
    for j in range(5):
        cx = x_ref[:, 64 + j:65 + j]
        cy = x_ref[:, 69 + j:70 + j]
        parts.append(cx * w1[0:1, :] + cy * w1[1:2, :] + b1_ref[...])
    a = jnp.concatenate(parts, axis=0)
    m = jnp.mean(a, axis=0, keepdims=True)
    v = jnp.mean((a - m) ** 2, axis=0, keepdims=True)
    h1 = jnp.maximum((a - m) * jax.lax.rsqrt(v + eps), 0.0)

    a2 = _dot(h1, w2_ref[...]) + b2_ref[...]
    m2 = jnp.mean(a2, axis=0, keepdims=True)
    v2 = jnp.mean((a2 - m2) ** 2, axis=0, keepdims=True)
    h2 = jnp.maximum((a2 - m2) * jax.lax.rsqrt(v2 + eps), 0.0)

    a3 = _dot(h2, w3_ref[...]) + b3_ref[...]
    m3 = jnp.mean(a3, axis=0, keepdims=True)
    v3 = jnp.mean((a3 - m3) ** 2, axis=0, keepdims=True)
    h3 = (a3 - m3) * jax.lax.rsqrt(v3 + eps)

    feat = h3[0:N_NEW]
    for j in range(1, 5):
        feat = jnp.maximum(feat, h3[j * N_NEW:(j + 1) * N_NEW])

    xx = (_dot(x_ref[:, :64], it1a_ref[...])
          + _dot(feat, it1b_ref[...])
          + it1_bias_ref[...])
    mx = jnp.mean(xx, axis=0, keepdims=True)
    vx = jnp.mean((xx - mx) ** 2, axis=0, keepdims=True)
    xx = jnp.maximum((xx - mx) * jax.lax.rsqrt(vx + eps), 0.0)
    xx = _dot(xx, it2_ref[...]) + it2_bias_ref[...]
    out_ref[...] = dnew_ref[...] * xx


def _bigmm_body(node_ref, edge_ref, h_ref,
                wr_ref, ur_ref, br_ref, wz_ref, uz_ref, bz_ref,
                wn_ref, un_ref, bn_ref,
                wno_ref, bno_ref, wne_ref, bne_ref,
                y_ref, hout_ref):
    i = pl.program_id(0)
    a_n = node_ref[...]
    a_e = edge_ref[...]
    h = h_ref[...]
    msum = _dot(a_n, h) + _dot(a_e, h)

    dcol_n = node_ref[:, pl.ds(i * _ROWS, _ROWS)]
    dcol_e = edge_ref[:, pl.ds(i * _ROWS, _ROWS)]
    ii = jax.lax.broadcasted_iota(jnp.int32, (_ROWS, _ROWS), 0)
    jj = jax.lax.broadcasted_iota(jnp.int32, (_ROWS, _ROWS), 1)
    mask = ii == jj
    d_n = jnp.sum(jnp.where(mask, dcol_n, 0.0), axis=1, keepdims=True)
    d_e = jnp.sum(jnp.where(mask, dcol_e, 0.0), axis=1, keepdims=True)

    ht = h_ref[pl.ds(i * _ROWS, _ROWS), :]
    r = jax.nn.sigmoid(
        _dot(msum, wr_ref[...])
        + _dot(ht, ur_ref[...])
        + br_ref[...])
    z = jax.nn.sigmoid(
        _dot(msum, wz_ref[...])
        + _dot(ht, uz_ref[...])
        + bz_ref[...])
    n = jnp.tanh(
        _dot(msum, wn_ref[...])
        + _dot(r * ht, un_ref[...])
        + bn_ref[...])
    h_out = (1.0 - z) * ht + z * n
    y = (d_n * (_dot(h_out, wno_ref[...])
                + bno_ref[...])
         + d_e * (_dot(h_out, wne_ref[...])
                  + bne_ref[...]))
    y_ref[...] = jax.nn.sigmoid(y)
    hout_ref[...] = h_out


def kernel(x, h_in, node_adj, edge_adj, pn_w1, pn_b1, pn_w2, pn_b2, pn_w3,
           pn_b3, it1_w, it1_b, it2_w, it2_b, gru_wr, gru_ur, gru_br,
           gru_wz, gru_uz, gru_bz, gru_wn, gru_un, gru_bn, out_node_w,
           out_node_b, out_edge_w, out_edge_b):
    f32 = jnp.float32

    nblk = N_NEW // 128
    off = N_OLD // 128
    d_new2d = pl.pallas_call(
        _diag_new_body,
        grid=(nblk,),
        in_specs=[pl.BlockSpec((128, 128), lambda i: (off + i, off + i))],
        out_specs=pl.BlockSpec((1, 1, 128), lambda i: (i, 0, 0)),
        out_shape=jax.ShapeDtypeStruct((nblk, 1, 128), f32),
    )(node_adj)
    d_new = d_new2d.reshape(N_NEW, 1)

    row = lambda v: v.reshape(1, -1)
    h_update = pl.pallas_call(
        _preproc_body,
        out_shape=jax.ShapeDtypeStruct((N_NEW, NHID), f32),
    )(x, pn_w1, row(pn_b1), pn_w2, row(pn_b2), pn_w3, row(pn_b3),
      it1_w[:64], it1_w[64:], row(it1_b), it2_w, row(it2_b), d_new)

    h = jnp.concatenate([h_in, h_update], axis=0)

    full = lambda shape: pl.BlockSpec(shape, lambda i: (0, 0))
    y, h_out = pl.pallas_call(
        _bigmm_body,
        grid=(_NTILE,),
        in_specs=[
            pl.BlockSpec((_ROWS, T), lambda i: (i, 0)),
            pl.BlockSpec((_ROWS, T), lambda i: (i, 0)),
            full((T, NHID)),
            full((NHID, NHID)), full((NHID, NHID)), full((1, NHID)),
            full((NHID, NHID)), full((NHID, NHID)), full((1, NHID)),
            full((NHID, NHID)), full((NHID, NHID)), full((1, NHID)),
            full((NHID, 1)), full((1, 1)), full((NHID, 1)), full((1, 1)),
        ],
        out_specs=[
            pl.BlockSpec((_ROWS, 1), lambda i: (i, 0)),
            pl.BlockSpec((_ROWS, NHID), lambda i: (i, 0)),
        ],
        out_shape=[
            jax.ShapeDtypeStruct((T, 1), f32),
            jax.ShapeDtypeStruct((T, NHID), f32),
        ],
        compiler_params=pltpu.CompilerParams(
            dimension_semantics=("arbitrary",),
        ),
    )(node_adj, edge_adj, h,
      gru_wr, gru_ur, row(gru_br), gru_wz, gru_uz, row(gru_bz),
      gru_wn, gru_un, row(gru_bn),
      out_node_w, out_node_b.reshape(1, 1),
      out_edge_w, out_edge_b.reshape(1, 1))
    return (y, h_out)
